# baseline (speedup 1.0000x reference)
import functools
import math

import jax
import jax.numpy as jnp
from jax import lax
from jax.experimental import pallas as pl
from jax.experimental.pallas import tpu as pltpu
from jax.experimental.pallas import tpu_sc as plsc

N = 50000
E = 800000
K = 5
NSPEC = 119
H = 128
RC = 6.0
SIGMA2_INV = 2.0

CHUNK = 256
NTILE = 16
NT_TILE = 196
NCHUNK = NT_TILE * NTILE
EPAD = NCHUNK * CHUNK
NACC = 50048
TRASH = 50040
ROWS_PER_TILE = NACC // NTILE
MW = 32
NPAD = 51200
BATOM = NPAD // 16

MU = [0.5 + 1.375 * k for k in range(K)]
PAIRS = [(0, 0), (0, 1), (0, 2), (1, 1), (1, 2), (2, 2)]
PAIR_W = [1.0, 2.0, 2.0, 1.0, 2.0, 1.0]
COS_COEF = [(-1.0) ** j / math.factorial(2 * j) for j in range(8)]


def _full(v, dtype=jnp.float32):
    return jnp.full((16,), v, dtype=dtype)


def _edge_kernel(rpad_hbm, ijhbm, out_hbm, acc, *bufs):
    c = lax.axis_index("c")
    s = lax.axis_index("s")
    ij = bufs[0:2]
    iis = bufs[2:4]
    riba = bufs[4:6]
    ribb = bufs[6:8]
    rjba = bufs[8:10]
    rjbb = bufs[10:12]
    paya = bufs[12:14]
    payb = bufs[14:16]
    zbuf = bufs[16]
    semi = bufs[17:19]
    semg = bufs[19:21]
    sems = bufs[21:23]

    z16 = jnp.zeros((16,), jnp.float32)
    t16 = jnp.full((16,), TRASH, jnp.int32)

    def zero_zbuf(r, _):
        zbuf[r, pl.ds(0, 16)] = z16
        zbuf[r, pl.ds(MW - 16, 16)] = z16
        return 0

    lax.fori_loop(0, 128, zero_zbuf, 0)

    for b in range(2):
        def zero_pay(r, _, b=b):
            paya[b][r, pl.ds(0, 16)] = z16
            paya[b][r, pl.ds(MW - 16, 16)] = z16
            payb[b][r, pl.ds(0, 16)] = z16
            payb[b][r, pl.ds(MW - 16, 16)] = z16
            return 0

        lax.fori_loop(0, 128, zero_pay, 0)
        for g in range(16):
            iis[b][g // 8, pl.ds((g % 8) * 16, 16)] = t16

    e16 = lax.iota(jnp.int32, 16)

    def issue_i(t, b):
        cc = s + NTILE * t
        pltpu.async_copy(ijhbm.at[cc], ij[b], semi[b])

    def wait_i(b):
        pltpu.make_async_copy(ijhbm.at[0], ij[b], semi[b]).wait()

    def issue_g(b):
        pltpu.async_copy(rpad_hbm.at[ij[b].at[0]], riba[b], semg[b])
        pltpu.async_copy(rpad_hbm.at[ij[b].at[1]], ribb[b], semg[b])
        pltpu.async_copy(rpad_hbm.at[ij[b].at[2]], rjba[b], semg[b])
        pltpu.async_copy(rpad_hbm.at[ij[b].at[3]], rjbb[b], semg[b])

    def wait_g(b):
        pltpu.make_async_copy(rpad_hbm.at[ij[b].at[0]], riba[b], semg[b]).wait()
        pltpu.make_async_copy(rpad_hbm.at[ij[b].at[1]], ribb[b], semg[b]).wait()
        pltpu.make_async_copy(rpad_hbm.at[ij[b].at[2]], rjba[b], semg[b]).wait()
        pltpu.make_async_copy(rpad_hbm.at[ij[b].at[3]], rjbb[b], semg[b]).wait()

    def issue_s(b):
        pltpu.async_copy(paya[b], acc.at[iis[b].at[0]], sems[b], add=True)
        pltpu.async_copy(payb[b], acc.at[iis[b].at[1]], sems[b], add=True)

    def wait_s(b):
        pltpu.make_async_copy(paya[b], acc.at[iis[b].at[0]], sems[b]).wait()
        pltpu.make_async_copy(payb[b], acc.at[iis[b].at[1]], sems[b]).wait()

    def compute(b):
        pb = b % 2
        for g in range(16):
            h = g // 8
            rows = e16 + ((g % 8) * 16)
            ri = riba[b] if h == 0 else ribb[b]
            rj = rjba[b] if h == 0 else rjbb[b]
            pay = paya[pb] if h == 0 else payb[pb]
            iis[pb][h, pl.ds((g % 8) * 16, 16)] = ij[b][h, pl.ds((g % 8) * 16,
                                                                 16)]
            xi = plsc.load_gather(ri, [rows, _full(0, jnp.int32)])
            yi = plsc.load_gather(ri, [rows, _full(1, jnp.int32)])
            zi = plsc.load_gather(ri, [rows, _full(2, jnp.int32)])
            xj = plsc.load_gather(rj, [rows, _full(0, jnp.int32)])
            yj = plsc.load_gather(rj, [rows, _full(1, jnp.int32)])
            zj = plsc.load_gather(rj, [rows, _full(2, jnp.int32)])
            dx = xj - xi
            dy = yj - yi
            dz = zj - zi
            d2 = dx * dx + dy * dy + dz * dz + 1e-12
            y = plsc.bitcast(
                jnp.int32(0x5F3759DF) - (plsc.bitcast(d2, jnp.int32) >> 1),
                jnp.float32)
            for _ in range(2):
                y = y * (1.5 - 0.5 * d2 * y * y)
            d = d2 * y
            ux = dx * y
            uy = dy * y
            uz = dz * y
            t_ang = jnp.minimum(d, RC) * (math.pi / RC)
            t2 = t_ang * t_ang
            ct = _full(COS_COEF[7])
            for j in range(6, -1, -1):
                ct = ct * t2 + COS_COEF[j]
            fc = 0.5 * (ct + 1.0)
            gk = [fc * jnp.exp((d - MU[k]) * (d - MU[k]) * (-SIGMA2_INV))
                  for k in range(K)]
            u = [ux, uy, uz]
            uab = [u[a] * u[b] for (a, b) in PAIRS]

            @pl.when(c == 0)
            def _():
                for k in range(K):
                    plsc.store_scatter(pay, [rows, _full(k, jnp.int32)],
                                       gk[k])
                for k in range(K):
                    for a in range(3):
                        plsc.store_scatter(
                            pay, [rows, _full(5 + 3 * k + a, jnp.int32)],
                            gk[k] * u[a])
                for p in range(5):
                    plsc.store_scatter(
                        pay, [rows, _full(20 + p, jnp.int32)],
                        gk[0] * uab[p])

            @pl.when(c == 1)
            def _():
                plsc.store_scatter(pay, [rows, _full(0, jnp.int32)],
                                   gk[0] * uab[5])
                for k in range(1, K):
                    for p in range(6):
                        plsc.store_scatter(
                            pay, [rows, _full(1 + 6 * (k - 1) + p,
                                              jnp.int32)],
                            gk[k] * uab[p])

    issue_i(jnp.int32(0), 0)
    issue_i(jnp.int32(1), 1)
    for b in range(2):
        issue_s(b)
    wait_i(0)
    issue_g(0)

    base = s * ROWS_PER_TILE
    for k in range(ROWS_PER_TILE // 128):
        pltpu.sync_copy(zbuf, acc.at[pl.ds(base + k * 128, 128)])
    rem = ROWS_PER_TILE % 128
    if rem:
        pltpu.sync_copy(zbuf.at[pl.ds(0, rem)],
                        acc.at[pl.ds(base + (ROWS_PER_TILE // 128) * 128, rem)])
    plsc.subcore_barrier()

    NT2 = NT_TILE // 2

    def pair_body(tt, _):
        for b in range(2):
            t = 2 * tt + b
            wait_g(b)
            wait_s(b)
            compute(b)
            issue_s(b)
            b1 = 1 - b

            def ahead():
                wait_i(b1)
                issue_g(b1)

            if b == 0:
                ahead()
            else:
                pl.when(tt < NT2 - 1)(ahead)

            @pl.when(tt < NT2 - 1)
            def _():
                issue_i(t + 2, b)
        return 0

    lax.fori_loop(0, NT2, pair_body, 0)
    for b in range(2):
        wait_s(b)
    plsc.subcore_barrier()

    pltpu.sync_copy(acc.at[pl.ds(base, ROWS_PER_TILE)],
                    out_hbm.at[c, pl.ds(base, ROWS_PER_TILE)])


def _edge_stage(rpad, ij_all):
    mesh = plsc.VectorSubcoreMesh(core_axis_name="c", subcore_axis_name="s")
    f = pl.kernel(
        _edge_kernel,
        out_type=jax.ShapeDtypeStruct((2, NACC, MW), jnp.float32),
        mesh=mesh,
        compiler_params=pltpu.CompilerParams(needs_layout_passes=False,
                                             use_tc_tiling_on_sc=False),
        scratch_types=(
            [pltpu.VMEM_SHARED((NACC, MW), jnp.float32)]
            + [pltpu.VMEM((4, 128), jnp.int32)] * 2
            + [pltpu.VMEM((2, 128), jnp.int32)] * 2
            + [pltpu.VMEM((128, 8), jnp.float32)] * 8
            + [pltpu.VMEM((128, MW), jnp.float32)] * 4
            + [pltpu.VMEM((128, MW), jnp.float32)]
            + [pltpu.SemaphoreType.DMA] * 6
        ),
    )
    return f(rpad, ij_all)


def _atom_kernel(msc_ref, z_ref, w1t_ref, b1_ref, w2t_ref, b2_ref, w3t_ref,
                 b3_ref, sc_ref, sh_ref, out_ref):
    i = pl.program_id(0)
    m0h = jnp.transpose(msc_ref[0])
    m1h = jnp.transpose(msc_ref[1])
    m0 = m0h[0:5]
    m1 = [m0h[5 + r] for r in range(15)]
    m2u = [m0h[20 + r] for r in range(5)] + [m1h[r] for r in range(25)]

    f1_rows = []
    for k in range(K):
        for l in range(K):
            acc = m1[3 * k] * m1[3 * l]
            for a in (1, 2):
                acc = acc + m1[3 * k + a] * m1[3 * l + a]
            f1_rows.append(acc)
    f2_rows = []
    for k in range(K):
        for l in range(K):
            acc = m2u[6 * k] * m2u[6 * l] * PAIR_W[0]
            for p in range(1, 6):
                acc = acc + m2u[6 * k + p] * m2u[6 * l + p] * PAIR_W[p]
            f2_rows.append(acc)

    gm = jnp.concatenate(
        [m0, jnp.stack(f1_rows), jnp.stack(f2_rows)], axis=0)

    def mm(a, b):
        return jax.lax.dot_general(a, b, (((1,), (0,)), ((), ())),
                                   preferred_element_type=jnp.float32)

    h1 = mm(w1t_ref[...], gm) + b1_ref[...]
    h1 = h1 * jax.nn.sigmoid(h1)
    h2 = mm(w2t_ref[...], h1) + b2_ref[...]
    h2 = h2 * jax.nn.sigmoid(h2)
    h3 = mm(w3t_ref[...], h2) + b3_ref[...]

    zrow = z_ref[...]
    oh = (lax.broadcasted_iota(jnp.int32, (128, BATOM), 0)
          == zrow).astype(jnp.float32)
    se = mm(sc_ref[...], oh)
    sh = mm(sh_ref[...], oh)
    e = h3 * se + sh

    @pl.when(i == 0)
    def _():
        out_ref[...] = jnp.zeros((1, 1), jnp.float32)

    out_ref[...] += jnp.sum(e).reshape(1, 1)


def _atom_stage(msc_pad, z_pad, w1t, b1c, w2t, b2c, w3t, b3c, scp, shp):
    grid = (NPAD // BATOM,)
    return pl.pallas_call(
        _atom_kernel,
        grid=grid,
        in_specs=[
            pl.BlockSpec((2, BATOM, MW), lambda i: (0, i, 0)),
            pl.BlockSpec((1, BATOM), lambda i: (0, i)),
            pl.BlockSpec((H, 55), lambda i: (0, 0)),
            pl.BlockSpec((H, 1), lambda i: (0, 0)),
            pl.BlockSpec((H, H), lambda i: (0, 0)),
            pl.BlockSpec((H, 1), lambda i: (0, 0)),
            pl.BlockSpec((1, H), lambda i: (0, 0)),
            pl.BlockSpec((1, 1), lambda i: (0, 0)),
            pl.BlockSpec((1, 128), lambda i: (0, 0)),
            pl.BlockSpec((1, 128), lambda i: (0, 0)),
        ],
        out_specs=pl.BlockSpec((1, 1), lambda i: (0, 0)),
        out_shape=jax.ShapeDtypeStruct((1, 1), jnp.float32),
    )(msc_pad, z_pad, w1t, b1c, w2t, b2c, w3t, b3c, scp, shp)


@jax.jit
def kernel(R, Z, idx, box, offsets, W1, b1, W2, b2, W3, b3, scale, shift):
    rpad = jnp.zeros((NACC, 8), jnp.float32).at[:N, :3].set(R)
    idx32 = idx.astype(jnp.int32)
    pad_i = jnp.full((1, EPAD - E), TRASH, jnp.int32)
    pad_j = jnp.zeros((1, EPAD - E), jnp.int32)
    pads = jnp.concatenate([pad_i, pad_j], axis=0)
    ij_all = (jnp.concatenate([idx32, pads], axis=1)
              .reshape(2, NCHUNK, 2, 128).transpose(1, 0, 2, 3)
              .reshape(NCHUNK, 4, 128))

    msc = _edge_stage(rpad, ij_all)
    msc_pad = jnp.pad(msc, ((0, 0), (0, NPAD - NACC), (0, 0)))

    z_pad = jnp.pad(Z.astype(jnp.int32), (0, NPAD - N),
                    constant_values=127).reshape(1, NPAD)
    w1t = W1.T
    w2t = W2.T
    w3t = W3.T
    b1c = b1.reshape(H, 1)
    b2c = b2.reshape(H, 1)
    b3c = b3.reshape(1, 1)
    scp = jnp.zeros((1, 128), jnp.float32).at[0, :NSPEC].set(scale)
    shp = jnp.zeros((1, 128), jnp.float32).at[0, :NSPEC].set(shift)

    out = _atom_stage(msc_pad, z_pad, w1t, b1c, w2t, b2c, w3t, b3c, scp, shp)
    return out[0, 0]


# speedup vs baseline: 1.2360x; 1.2360x over previous
import functools
import math

import jax
import jax.numpy as jnp
from jax import lax
from jax.experimental import pallas as pl
from jax.experimental.pallas import tpu as pltpu
from jax.experimental.pallas import tpu_sc as plsc

N = 50000
E = 800000
K = 5
NSPEC = 119
H = 128
RC = 6.0
SIGMA2_INV = 2.0

CHUNK = 128
NTILE = 16
NT_TILE = 392
NCHUNK = NT_TILE * NTILE
EPAD = NCHUNK * CHUNK
NACC = 50048
TRASH = 50040
ROWS_PER_TILE = NACC // NTILE
MW = 32
NPAD = 51200
BATOM = NPAD // 16

MU = [0.5 + 1.375 * k for k in range(K)]
PAIRS = [(0, 0), (0, 1), (0, 2), (1, 1), (1, 2), (2, 2)]
PAIR_W = [1.0, 2.0, 2.0, 1.0, 2.0, 1.0]
COS_COEF = [(-1.0) ** j / math.factorial(2 * j) for j in range(8)]


def _full(v, dtype=jnp.float32):
    return jnp.full((16,), v, dtype=dtype)


def _edge_kernel(rpad_hbm, ijhbm, out_hbm, acc, *bufs):
    c = lax.axis_index("c")
    s = lax.axis_index("s")
    ij = bufs[0:4]
    iis = bufs[4:8]
    rib = bufs[8:12]
    rjb = bufs[12:16]
    pay = bufs[16:20]
    zbuf = bufs[20]
    semi = bufs[21:25]
    semg = bufs[25:29]
    sems = bufs[29:33]

    z16 = jnp.zeros((16,), jnp.float32)
    t16 = jnp.full((16,), TRASH, jnp.int32)

    def zero_zbuf(r, _):
        zbuf[r, pl.ds(0, 16)] = z16
        zbuf[r, pl.ds(MW - 16, 16)] = z16
        return 0

    lax.fori_loop(0, 128, zero_zbuf, 0)

    for b in range(4):
        def zero_pay(r, _, b=b):
            pay[b][r, pl.ds(0, 16)] = z16
            pay[b][r, pl.ds(MW - 16, 16)] = z16
            return 0

        lax.fori_loop(0, CHUNK, zero_pay, 0)
        for g in range(CHUNK // 16):
            iis[b][0, pl.ds(g * 16, 16)] = t16

    e16 = lax.iota(jnp.int32, 16)

    def issue_i(t, b):
        cc = s + NTILE * t
        pltpu.async_copy(ijhbm.at[cc], ij[b], semi[b])

    def wait_i(b):
        pltpu.make_async_copy(ijhbm.at[0], ij[b], semi[b]).wait()

    def issue_g(b):
        pltpu.async_copy(rpad_hbm.at[ij[b].at[0]], rib[b], semg[b])
        pltpu.async_copy(rpad_hbm.at[ij[b].at[1]], rjb[b], semg[b])

    def wait_g(b):
        pltpu.make_async_copy(rpad_hbm.at[ij[b].at[0]], rib[b], semg[b]).wait()
        pltpu.make_async_copy(rpad_hbm.at[ij[b].at[1]], rjb[b], semg[b]).wait()

    def issue_s(b):
        pltpu.async_copy(pay[b], acc.at[iis[b].at[0]], sems[b], add=True)

    def wait_s(b):
        pltpu.make_async_copy(pay[b], acc.at[iis[b].at[0]], sems[b]).wait()

    def compute(b):
        for g in range(CHUNK // 16):
            rows = e16 + (g * 16)
            iis[b][0, pl.ds(g * 16, 16)] = ij[b][0, pl.ds(g * 16, 16)]
            xi = plsc.load_gather(rib[b], [rows, _full(0, jnp.int32)])
            yi = plsc.load_gather(rib[b], [rows, _full(1, jnp.int32)])
            zi = plsc.load_gather(rib[b], [rows, _full(2, jnp.int32)])
            xj = plsc.load_gather(rjb[b], [rows, _full(0, jnp.int32)])
            yj = plsc.load_gather(rjb[b], [rows, _full(1, jnp.int32)])
            zj = plsc.load_gather(rjb[b], [rows, _full(2, jnp.int32)])
            dx = xj - xi
            dy = yj - yi
            dz = zj - zi
            d2 = dx * dx + dy * dy + dz * dz + 1e-12
            y = plsc.bitcast(
                jnp.int32(0x5F3759DF) - (plsc.bitcast(d2, jnp.int32) >> 1),
                jnp.float32)
            for _ in range(2):
                y = y * (1.5 - 0.5 * d2 * y * y)
            d = d2 * y
            ux = dx * y
            uy = dy * y
            uz = dz * y
            t_ang = jnp.minimum(d, RC) * (math.pi / RC)
            t2 = t_ang * t_ang
            ct = _full(COS_COEF[7])
            for j in range(6, -1, -1):
                ct = ct * t2 + COS_COEF[j]
            fc = 0.5 * (ct + 1.0)
            gk = [fc * jnp.exp((d - MU[k]) * (d - MU[k]) * (-SIGMA2_INV))
                  for k in range(K)]
            u = [ux, uy, uz]
            uab = [u[a] * u[b] for (a, b) in PAIRS]

            @pl.when(c == 0)
            def _():
                for k in range(K):
                    plsc.store_scatter(pay[b], [rows, _full(k, jnp.int32)],
                                       gk[k])
                for k in range(K):
                    for a in range(3):
                        plsc.store_scatter(
                            pay[b], [rows, _full(5 + 3 * k + a, jnp.int32)],
                            gk[k] * u[a])
                for p in range(5):
                    plsc.store_scatter(
                        pay[b], [rows, _full(20 + p, jnp.int32)],
                        gk[0] * uab[p])

            @pl.when(c == 1)
            def _():
                plsc.store_scatter(pay[b], [rows, _full(0, jnp.int32)],
                                   gk[0] * uab[5])
                for k in range(1, K):
                    for p in range(6):
                        plsc.store_scatter(
                            pay[b], [rows, _full(1 + 6 * (k - 1) + p,
                                                 jnp.int32)],
                            gk[k] * uab[p])

    issue_i(jnp.int32(0), 0)
    issue_i(jnp.int32(1), 1)
    issue_i(jnp.int32(2), 2)
    for b in range(4):
        issue_s(b)
    wait_i(0)
    issue_g(0)
    issue_i(jnp.int32(3), 3)
    wait_i(1)
    issue_g(1)
    wait_i(2)
    issue_g(2)

    base = s * ROWS_PER_TILE
    for k in range(ROWS_PER_TILE // 128):
        pltpu.sync_copy(zbuf, acc.at[pl.ds(base + k * 128, 128)])
    rem = ROWS_PER_TILE % 128
    if rem:
        pltpu.sync_copy(zbuf.at[pl.ds(0, rem)],
                        acc.at[pl.ds(base + (ROWS_PER_TILE // 128) * 128, rem)])
    plsc.subcore_barrier()

    NT4 = NT_TILE // 4

    def quad_body(tt, _):
        for b in range(4):
            t = 4 * tt + b
            wait_g(b)
            wait_s(b)
            compute(b)
            issue_s(b)
            b3 = (b + 3) % 4

            def ahead():
                wait_i(b3)
                issue_g(b3)

            if b == 0:
                ahead()
            else:
                pl.when(tt < NT4 - 1)(ahead)

            @pl.when(tt < NT4 - 1)
            def _():
                issue_i(t + 4, b)
        return 0

    lax.fori_loop(0, NT4, quad_body, 0)
    for b in range(4):
        wait_s(b)
    plsc.subcore_barrier()

    pltpu.sync_copy(acc.at[pl.ds(base, ROWS_PER_TILE)],
                    out_hbm.at[c, pl.ds(base, ROWS_PER_TILE)])


def _edge_stage(rpad, ij_all):
    mesh = plsc.VectorSubcoreMesh(core_axis_name="c", subcore_axis_name="s")
    f = pl.kernel(
        _edge_kernel,
        out_type=jax.ShapeDtypeStruct((2, NACC, MW), jnp.float32),
        mesh=mesh,
        compiler_params=pltpu.CompilerParams(needs_layout_passes=False,
                                             use_tc_tiling_on_sc=False),
        scratch_types=(
            [pltpu.VMEM_SHARED((NACC, MW), jnp.float32)]
            + [pltpu.VMEM((2, CHUNK), jnp.int32)] * 4
            + [pltpu.VMEM((1, CHUNK), jnp.int32)] * 4
            + [pltpu.VMEM((CHUNK, 8), jnp.float32)] * 8
            + [pltpu.VMEM((CHUNK, MW), jnp.float32)] * 4
            + [pltpu.VMEM((128, MW), jnp.float32)]
            + [pltpu.SemaphoreType.DMA] * 12
        ),
    )
    return f(rpad, ij_all)


def _atom_kernel(msc_ref, z_ref, w1t_ref, b1_ref, w2t_ref, b2_ref, w3t_ref,
                 b3_ref, sc_ref, sh_ref, out_ref):
    i = pl.program_id(0)
    m0h = jnp.transpose(msc_ref[0])
    m1h = jnp.transpose(msc_ref[1])
    m0 = m0h[0:5]
    m1 = [m0h[5 + r] for r in range(15)]
    m2u = [m0h[20 + r] for r in range(5)] + [m1h[r] for r in range(25)]

    f1_rows = []
    for k in range(K):
        for l in range(K):
            acc = m1[3 * k] * m1[3 * l]
            for a in (1, 2):
                acc = acc + m1[3 * k + a] * m1[3 * l + a]
            f1_rows.append(acc)
    f2_rows = []
    for k in range(K):
        for l in range(K):
            acc = m2u[6 * k] * m2u[6 * l] * PAIR_W[0]
            for p in range(1, 6):
                acc = acc + m2u[6 * k + p] * m2u[6 * l + p] * PAIR_W[p]
            f2_rows.append(acc)

    gm = jnp.concatenate(
        [m0, jnp.stack(f1_rows), jnp.stack(f2_rows)], axis=0)

    def mm(a, b):
        return jax.lax.dot_general(a, b, (((1,), (0,)), ((), ())),
                                   preferred_element_type=jnp.float32)

    h1 = mm(w1t_ref[...], gm) + b1_ref[...]
    h1 = h1 * jax.nn.sigmoid(h1)
    h2 = mm(w2t_ref[...], h1) + b2_ref[...]
    h2 = h2 * jax.nn.sigmoid(h2)
    h3 = mm(w3t_ref[...], h2) + b3_ref[...]

    zrow = z_ref[...]
    oh = (lax.broadcasted_iota(jnp.int32, (128, BATOM), 0)
          == zrow).astype(jnp.float32)
    se = mm(sc_ref[...], oh)
    sh = mm(sh_ref[...], oh)
    e = h3 * se + sh

    @pl.when(i == 0)
    def _():
        out_ref[...] = jnp.zeros((1, 1), jnp.float32)

    out_ref[...] += jnp.sum(e).reshape(1, 1)


def _atom_stage(msc_pad, z_pad, w1t, b1c, w2t, b2c, w3t, b3c, scp, shp):
    grid = (NPAD // BATOM,)
    return pl.pallas_call(
        _atom_kernel,
        grid=grid,
        in_specs=[
            pl.BlockSpec((2, BATOM, MW), lambda i: (0, i, 0)),
            pl.BlockSpec((1, BATOM), lambda i: (0, i)),
            pl.BlockSpec((H, 55), lambda i: (0, 0)),
            pl.BlockSpec((H, 1), lambda i: (0, 0)),
            pl.BlockSpec((H, H), lambda i: (0, 0)),
            pl.BlockSpec((H, 1), lambda i: (0, 0)),
            pl.BlockSpec((1, H), lambda i: (0, 0)),
            pl.BlockSpec((1, 1), lambda i: (0, 0)),
            pl.BlockSpec((1, 128), lambda i: (0, 0)),
            pl.BlockSpec((1, 128), lambda i: (0, 0)),
        ],
        out_specs=pl.BlockSpec((1, 1), lambda i: (0, 0)),
        out_shape=jax.ShapeDtypeStruct((1, 1), jnp.float32),
    )(msc_pad, z_pad, w1t, b1c, w2t, b2c, w3t, b3c, scp, shp)


@jax.jit
def kernel(R, Z, idx, box, offsets, W1, b1, W2, b2, W3, b3, scale, shift):
    rpad = jnp.zeros((NACC, 8), jnp.float32).at[:N, :3].set(R)
    idx32 = idx.astype(jnp.int32)
    pad_i = jnp.full((1, EPAD - E), TRASH, jnp.int32)
    pad_j = jnp.zeros((1, EPAD - E), jnp.int32)
    pads = jnp.concatenate([pad_i, pad_j], axis=0)
    ij_all = (jnp.concatenate([idx32, pads], axis=1)
              .reshape(2, NCHUNK, CHUNK).transpose(1, 0, 2))

    msc = _edge_stage(rpad, ij_all)
    msc_pad = jnp.pad(msc, ((0, 0), (0, NPAD - NACC), (0, 0)))

    z_pad = jnp.pad(Z.astype(jnp.int32), (0, NPAD - N),
                    constant_values=127).reshape(1, NPAD)
    w1t = W1.T
    w2t = W2.T
    w3t = W3.T
    b1c = b1.reshape(H, 1)
    b2c = b2.reshape(H, 1)
    b3c = b3.reshape(1, 1)
    scp = jnp.zeros((1, 128), jnp.float32).at[0, :NSPEC].set(scale)
    shp = jnp.zeros((1, 128), jnp.float32).at[0, :NSPEC].set(shift)

    out = _atom_stage(msc_pad, z_pad, w1t, b1c, w2t, b2c, w3t, b3c, scp, shp)
    return out[0, 0]
